# baseline (speedup 1.0000x reference)
import functools

import jax
import jax.numpy as jnp
from jax import lax
from jax.experimental import pallas as pl
from jax.experimental.pallas import tpu as pltpu
from jax.experimental.pallas import tpu_sc as plsc

N_NODES = 10000
N_SRC = 5000
N_EDGES = 320000
HIDDEN = 128
HEADS = 8
DH = 16

NW = 32
EPW = 10240
EPAD = NW * EPW
CH = 128
NCH = EPW // CH
PD = 5120
RPT = PD // 16
NEG_SLOPE = 0.2


def _dense_body(x_ref, w_ref, alr_ref, feat_ref, elr_ref):
    feat = jnp.dot(x_ref[...], w_ref[...], preferred_element_type=jnp.float32)
    feat_ref[...] = feat
    elr_ref[...] = jnp.dot(feat, alr_ref[...], preferred_element_type=jnp.float32)


def _dense(x, W, ALR):
    blk = 1000
    return pl.pallas_call(
        _dense_body,
        grid=(N_NODES // blk,),
        in_specs=[
            pl.BlockSpec((blk, HIDDEN), lambda i: (i, 0)),
            pl.BlockSpec((HIDDEN, HIDDEN), lambda i: (0, 0)),
            pl.BlockSpec((HIDDEN, HIDDEN), lambda i: (0, 0)),
        ],
        out_specs=[
            pl.BlockSpec((blk, HIDDEN), lambda i: (i, 0)),
            pl.BlockSpec((blk, HIDDEN), lambda i: (i, 0)),
        ],
        out_shape=[
            jax.ShapeDtypeStruct((N_NODES, HIDDEN), jnp.float32),
            jax.ShapeDtypeStruct((N_NODES, HIDDEN), jnp.float32),
        ],
    )(x, W, ALR)


def _edge_body(ei_ref, s_ref, d_ref):
    i = pl.program_id(0)
    f = (i * 16384
         + lax.broadcasted_iota(jnp.int32, (128, 128), 0) * 128
         + lax.broadcasted_iota(jnp.int32, (128, 128), 1))
    real = f < N_EDGES
    e0 = ei_ref[0]
    e1 = ei_ref[1]
    s_ref[...] = jnp.where(real, e0 % N_SRC, f % N_SRC)
    d_ref[...] = jnp.where(real, e1 % N_SRC, N_SRC + f % (PD - N_SRC))


def _edges(ei3):
    rows = EPAD // 128
    return pl.pallas_call(
        _edge_body,
        grid=(rows // 128,),
        in_specs=[pl.BlockSpec((2, 128, 128), lambda i: (0, i, 0))],
        out_specs=[
            pl.BlockSpec((128, 128), lambda i: (i, 0)),
            pl.BlockSpec((128, 128), lambda i: (i, 0)),
        ],
        out_shape=[
            jax.ShapeDtypeStruct((rows, 128), jnp.int32),
            jax.ShapeDtypeStruct((rows, 128), jnp.int32),
        ],
    )(ei3)


def _sc_body(elr_hbm, feat_hbm, sidx_hbm, didx_hbm,
             rst_out, denx_out,
             sidx_v, didx_v, featbuf, msgbuf, wbuf, elbuf, erbuf, elbuf2,
             erbuf2, den_v, el_sh, er_sh, rst_sh, den_sh,
             seme, semf, semd, semr):
    cid = lax.axis_index("c")
    sid = lax.axis_index("s")

    w = cid * 16 + sid
    pltpu.sync_copy(sidx_hbm.at[pl.ds(w * NCH, NCH)], sidx_v)
    pltpu.sync_copy(didx_hbm.at[pl.ds(w * NCH, NCH)], didx_v)

    rows_fill = N_SRC // 8

    @pl.when(sid < 8)
    def _fill_el():
        pltpu.sync_copy(
            elr_hbm.at[pl.ds(sid * rows_fill, rows_fill), pl.ds(0, HEADS)],
            el_sh.at[pl.ds(sid * rows_fill, rows_fill)])

    @pl.when(sid >= 8)
    def _fill_er():
        pltpu.sync_copy(
            elr_hbm.at[pl.ds(N_SRC + (sid - 8) * rows_fill, rows_fill),
                       pl.ds(HEADS, HEADS)],
            er_sh.at[pl.ds((sid - 8) * rows_fill, rows_fill)])

    zf = jnp.zeros((16,), jnp.float32)
    i16 = lax.broadcasted_iota(jnp.int32, (16,), 0)

    @plsc.parallel_loop(0, CH, 1, unroll=4)
    def _zrow(i):
        for j in range(HIDDEN // 16):
            featbuf[i, pl.ds(j * 16, 16)] = zf

    @plsc.parallel_loop(0, CH * HEADS // 16, 1, unroll=4)
    def _zw(k):
        kk = k * 16 + i16
        plsc.store_scatter(wbuf, [kk // HEADS, kk % HEADS], zf)

    base = sid * RPT
    pltpu.sync_copy(featbuf, rst_sh.at[pl.ds(base, CH)])
    pltpu.sync_copy(featbuf, rst_sh.at[pl.ds(base + CH, CH)])
    pltpu.sync_copy(featbuf.at[pl.ds(0, RPT - 2 * CH)],
                    rst_sh.at[pl.ds(base + 2 * CH, RPT - 2 * CH)])
    pltpu.sync_copy(wbuf, den_sh.at[pl.ds(base, CH)])
    pltpu.sync_copy(wbuf, den_sh.at[pl.ds(base + CH, CH)])
    pltpu.sync_copy(wbuf.at[pl.ds(0, RPT - 2 * CH)],
                    den_sh.at[pl.ds(base + 2 * CH, RPT - 2 * CH)])

    @pl.when(sid == 15)
    def _zero_er_pad():
        pltpu.sync_copy(wbuf.at[pl.ds(0, PD - N_SRC)],
                        er_sh.at[pl.ds(N_SRC, PD - N_SRC)])

    plsc.subcore_barrier()

    elbufs = (elbuf, elbuf2)
    erbufs = (erbuf, erbuf2)

    pltpu.async_copy(el_sh.at[sidx_v.at[0]], elbufs[0], seme)
    pltpu.async_copy(er_sh.at[didx_v.at[0]], erbufs[0], seme)
    pltpu.async_copy(feat_hbm.at[sidx_v.at[0]], featbuf, semf)

    def _chunk(c, par):
        elb, erb = elbufs[par], erbufs[par]
        pltpu.make_async_copy(el_sh.at[sidx_v.at[c]], elb, seme).wait()
        pltpu.make_async_copy(er_sh.at[didx_v.at[c]], erb, seme).wait()

        @pl.when(c >= 1)
        def _wait_den():
            pltpu.make_async_copy(wbuf, den_sh.at[didx_v.at[0]], semd).wait()

        @plsc.parallel_loop(0, CH * HEADS // 16, 1, unroll=4)
        def _wcalc(p):
            fl = p * 16 + i16
            rows = fl // HEADS
            cols = fl % HEADS
            z = (plsc.load_gather(elb, [rows, cols])
                 + plsc.load_gather(erb, [rows, cols]))
            wv = jnp.exp(jnp.where(z > 0, z, z * NEG_SLOPE))
            plsc.store_scatter(wbuf, [rows, cols], wv)

        pltpu.async_copy(wbuf, den_sh.at[didx_v.at[c]], semd, add=True)

        @pl.when(c + 1 < NCH)
        def _prefetch():
            pltpu.async_copy(el_sh.at[sidx_v.at[c + 1]], elbufs[1 - par], seme)
            pltpu.async_copy(er_sh.at[didx_v.at[c + 1]], erbufs[1 - par], seme)

        pltpu.make_async_copy(feat_hbm.at[sidx_v.at[c]], featbuf, semf).wait()

        @pl.when(c >= 1)
        def _wait_rst():
            pltpu.make_async_copy(msgbuf, rst_sh.at[didx_v.at[0]], semr).wait()

        @plsc.parallel_loop(0, CH // 2, 1, unroll=4)
        def _scale(p):
            fl = p * 16 + i16
            wrow = plsc.load_gather(wbuf, [fl // HEADS, fl % HEADS])
            for h in range(HEADS):
                sl = pl.ds(h * DH, DH)
                msgbuf[2 * p, sl] = featbuf[2 * p, sl] * wrow[h]
                msgbuf[2 * p + 1, sl] = featbuf[2 * p + 1, sl] * wrow[HEADS + h]

        @pl.when(c + 1 < NCH)
        def _prefetch_feat():
            pltpu.async_copy(feat_hbm.at[sidx_v.at[c + 1]], featbuf, semf)

        pltpu.async_copy(msgbuf, rst_sh.at[didx_v.at[c]], semr, add=True)

    def _chunk2(cc, _):
        _chunk(2 * cc, 0)
        _chunk(2 * cc + 1, 1)
        return 0

    lax.fori_loop(0, NCH // 2, _chunk2, 0)
    pltpu.make_async_copy(wbuf, den_sh.at[didx_v.at[0]], semd).wait()
    pltpu.make_async_copy(msgbuf, rst_sh.at[didx_v.at[0]], semr).wait()
    plsc.subcore_barrier()

    pltpu.sync_copy(rst_sh.at[pl.ds(base, RPT)],
                    rst_out.at[cid, pl.ds(base, RPT)])

    pltpu.sync_copy(den_sh.at[pl.ds(base, RPT)], den_v)
    for g in range(3):
        rows = CH if g < 2 else RPT - 2 * CH

        @plsc.parallel_loop(0, rows, 1, unroll=2)
        def _exp(r):
            for h in range(HEADS):
                val = plsc.load_gather(
                    den_v, [jnp.full((16,), g * CH + r, jnp.int32),
                            jnp.full((16,), h, jnp.int32)])
                featbuf[r, pl.ds(h * DH, DH)] = val
        pltpu.sync_copy(featbuf.at[pl.ds(0, rows)],
                        denx_out.at[cid, pl.ds(base + g * CH, rows)])


def _sc_pass(elr, feat, sidx, didx):
    mesh = plsc.VectorSubcoreMesh(core_axis_name="c", subcore_axis_name="s")
    fn = functools.partial(
        pl.kernel,
        mesh=mesh,
        compiler_params=pltpu.CompilerParams(needs_layout_passes=False,
                                             use_tc_tiling_on_sc=False),
        out_type=[
            jax.ShapeDtypeStruct((2, PD, HIDDEN), jnp.float32),
            jax.ShapeDtypeStruct((2, PD, HIDDEN), jnp.float32),
        ],
        scratch_types=[
            pltpu.VMEM((NCH, CH), jnp.int32),
            pltpu.VMEM((NCH, CH), jnp.int32),
            pltpu.VMEM((CH, HIDDEN), jnp.float32),
            pltpu.VMEM((CH, HIDDEN), jnp.float32),
            pltpu.VMEM((CH, HEADS), jnp.float32),
            pltpu.VMEM((CH, HEADS), jnp.float32),
            pltpu.VMEM((CH, HEADS), jnp.float32),
            pltpu.VMEM((CH, HEADS), jnp.float32),
            pltpu.VMEM((CH, HEADS), jnp.float32),
            pltpu.VMEM((RPT, HEADS), jnp.float32),
            pltpu.VMEM_SHARED((N_SRC, HEADS), jnp.float32),
            pltpu.VMEM_SHARED((PD, HEADS), jnp.float32),
            pltpu.VMEM_SHARED((PD, HIDDEN), jnp.float32),
            pltpu.VMEM_SHARED((PD, HEADS), jnp.float32),
            pltpu.SemaphoreType.DMA,
            pltpu.SemaphoreType.DMA,
            pltpu.SemaphoreType.DMA,
            pltpu.SemaphoreType.DMA,
        ],
    )(_sc_body)
    return fn(elr, feat, sidx, didx)


def _combine_body(r0_ref, r1_ref, d0_ref, d1_ref, elr_ref, feat_ref,
                  bias_ref, out_ref):
    el = elr_ref[:, 0:HEADS]
    er = elr_ref[:, HEADS:2 * HEADS]
    zs = el + er
    wself = jnp.exp(jnp.where(zs > 0, zs, zs * NEG_SLOPE))
    b = bias_ref[...]
    for h in range(HEADS):
        sl = slice(h * DH, (h + 1) * DH)
        wcol = wself[:, h:h + 1]
        num = r0_ref[0][:, sl] + r1_ref[0][:, sl] + wcol * feat_ref[:, sl]
        den = d0_ref[0][:, sl] + d1_ref[0][:, sl] + wcol + 1e-9
        out_ref[:, sl] = num / den + b[:, sl]


def _combine(rstp, denxp, elr, feat, bias2d):
    blk = 1000
    return pl.pallas_call(
        _combine_body,
        grid=(N_SRC // blk,),
        in_specs=[
            pl.BlockSpec((1, blk, HIDDEN), lambda i: (0, i, 0)),
            pl.BlockSpec((1, blk, HIDDEN), lambda i: (1, i, 0)),
            pl.BlockSpec((1, blk, HIDDEN), lambda i: (0, i, 0)),
            pl.BlockSpec((1, blk, HIDDEN), lambda i: (1, i, 0)),
            pl.BlockSpec((blk, HIDDEN), lambda i: (i + N_SRC // blk, 0)),
            pl.BlockSpec((blk, HIDDEN), lambda i: (i + N_SRC // blk, 0)),
            pl.BlockSpec((1, HIDDEN), lambda i: (0, 0)),
        ],
        out_specs=pl.BlockSpec((blk, HIDDEN), lambda i: (i, 0)),
        out_shape=jax.ShapeDtypeStruct((N_SRC, HIDDEN), jnp.float32),
    )(rstp, rstp, denxp, denxp, elr, feat, bias2d)


def kernel(x, edge_index, W, attn_l, attn_r, bias):
    alf = attn_l.reshape(HIDDEN)
    arf = attn_r.reshape(HIDDEN)
    sel = (jnp.arange(HIDDEN)[:, None] // DH
           == jnp.arange(HEADS)[None, :]).astype(jnp.float32)
    ALR = jnp.concatenate([alf[:, None] * sel, arf[:, None] * sel], axis=1)
    ALR = jnp.pad(ALR, ((0, 0), (0, HIDDEN - 2 * HEADS)))

    feat, elr = _dense(x, W, ALR)

    ei3 = edge_index.astype(jnp.int32).reshape(2, N_EDGES // 128, 128)
    sidx, didx = _edges(ei3)

    rstp, denxp = _sc_pass(elr, feat, sidx, didx)

    bias2d = bias.reshape(1, HIDDEN)
    return _combine(rstp, denxp, elr, feat, bias2d)


# speedup vs baseline: 1.0795x; 1.0795x over previous
import functools

import jax
import jax.numpy as jnp
from jax import lax
from jax.experimental import pallas as pl
from jax.experimental.pallas import tpu as pltpu
from jax.experimental.pallas import tpu_sc as plsc

N_NODES = 10000
N_SRC = 5000
N_EDGES = 320000
HIDDEN = 128
HEADS = 8
DH = 16

NW = 32
EPW = 10240
EPAD = NW * EPW
CH = 128
NCH = EPW // CH
PD = 5120
RPT = PD // 16
NEG_SLOPE = 0.2


def _dense_body(x_ref, w_ref, alr_ref, feat_ref, elr_ref):
    feat = jnp.dot(x_ref[...], w_ref[...], preferred_element_type=jnp.float32)
    feat_ref[...] = feat
    elr_ref[...] = jnp.dot(feat, alr_ref[...], preferred_element_type=jnp.float32)


def _dense(x, W, ALR):
    blk = 1000
    return pl.pallas_call(
        _dense_body,
        grid=(N_NODES // blk,),
        in_specs=[
            pl.BlockSpec((blk, HIDDEN), lambda i: (i, 0)),
            pl.BlockSpec((HIDDEN, HIDDEN), lambda i: (0, 0)),
            pl.BlockSpec((HIDDEN, HIDDEN), lambda i: (0, 0)),
        ],
        out_specs=[
            pl.BlockSpec((blk, HIDDEN), lambda i: (i, 0)),
            pl.BlockSpec((blk, HIDDEN), lambda i: (i, 0)),
        ],
        out_shape=[
            jax.ShapeDtypeStruct((N_NODES, HIDDEN), jnp.float32),
            jax.ShapeDtypeStruct((N_NODES, HIDDEN), jnp.float32),
        ],
    )(x, W, ALR)


def _edge_body(ei_ref, s_ref, d_ref):
    i = pl.program_id(0)
    f = (i * 16384
         + lax.broadcasted_iota(jnp.int32, (128, 128), 0) * 128
         + lax.broadcasted_iota(jnp.int32, (128, 128), 1))
    real = f < N_EDGES
    e0 = ei_ref[0]
    e1 = ei_ref[1]
    s_ref[...] = jnp.where(real, e0 % N_SRC, f % N_SRC)
    d_ref[...] = jnp.where(real, e1 % N_SRC, N_SRC + f % (PD - N_SRC))


def _edges(ei3):
    rows = EPAD // 128
    return pl.pallas_call(
        _edge_body,
        grid=(rows // 128,),
        in_specs=[pl.BlockSpec((2, 128, 128), lambda i: (0, i, 0))],
        out_specs=[
            pl.BlockSpec((128, 128), lambda i: (i, 0)),
            pl.BlockSpec((128, 128), lambda i: (i, 0)),
        ],
        out_shape=[
            jax.ShapeDtypeStruct((rows, 128), jnp.int32),
            jax.ShapeDtypeStruct((rows, 128), jnp.int32),
        ],
    )(ei3)


def _sc_body(elr_hbm, feat_hbm, sidx_hbm, didx_hbm,
             rst_out, denx_out,
             sidx_v, didx_v, featbuf, msgbuf, wbuf, elbuf, erbuf, elbuf2,
             erbuf2, den_v, el_sh, er_sh, rst_sh, den_sh,
             seme, semf, semd, semr):
    cid = lax.axis_index("c")
    sid = lax.axis_index("s")

    w = cid * 16 + sid
    pltpu.sync_copy(sidx_hbm.at[pl.ds(w * NCH, NCH)], sidx_v)
    pltpu.sync_copy(didx_hbm.at[pl.ds(w * NCH, NCH)], didx_v)

    rows_fill = N_SRC // 8

    @pl.when(sid < 8)
    def _fill_el():
        pltpu.sync_copy(
            elr_hbm.at[pl.ds(sid * rows_fill, rows_fill), pl.ds(0, HEADS)],
            el_sh.at[pl.ds(sid * rows_fill, rows_fill)])

    @pl.when(sid >= 8)
    def _fill_er():
        pltpu.sync_copy(
            elr_hbm.at[pl.ds(N_SRC + (sid - 8) * rows_fill, rows_fill),
                       pl.ds(HEADS, HEADS)],
            er_sh.at[pl.ds((sid - 8) * rows_fill, rows_fill)])

    zf = jnp.zeros((16,), jnp.float32)
    i16 = lax.broadcasted_iota(jnp.int32, (16,), 0)

    @plsc.parallel_loop(0, CH, 1, unroll=4)
    def _zrow(i):
        for j in range(HIDDEN // 16):
            featbuf[i, pl.ds(j * 16, 16)] = zf

    @plsc.parallel_loop(0, CH * HEADS // 16, 1, unroll=4)
    def _zw(k):
        kk = k * 16 + i16
        plsc.store_scatter(wbuf, [kk // HEADS, kk % HEADS], zf)

    base = sid * RPT
    pltpu.sync_copy(featbuf, rst_sh.at[pl.ds(base, CH)])
    pltpu.sync_copy(featbuf, rst_sh.at[pl.ds(base + CH, CH)])
    pltpu.sync_copy(featbuf.at[pl.ds(0, RPT - 2 * CH)],
                    rst_sh.at[pl.ds(base + 2 * CH, RPT - 2 * CH)])
    pltpu.sync_copy(wbuf, den_sh.at[pl.ds(base, CH)])
    pltpu.sync_copy(wbuf, den_sh.at[pl.ds(base + CH, CH)])
    pltpu.sync_copy(wbuf.at[pl.ds(0, RPT - 2 * CH)],
                    den_sh.at[pl.ds(base + 2 * CH, RPT - 2 * CH)])

    @pl.when(sid == 15)
    def _zero_er_pad():
        pltpu.sync_copy(wbuf.at[pl.ds(0, PD - N_SRC)],
                        er_sh.at[pl.ds(N_SRC, PD - N_SRC)])

    plsc.subcore_barrier()

    elbufs = (elbuf, elbuf2)
    erbufs = (erbuf, erbuf2)

    pltpu.async_copy(el_sh.at[sidx_v.at[0]], elbufs[0], seme)
    pltpu.async_copy(er_sh.at[didx_v.at[0]], erbufs[0], seme)
    pltpu.async_copy(feat_hbm.at[sidx_v.at[0]], featbuf, semf)

    def _chunk(c, par):
        elb, erb = elbufs[par], erbufs[par]
        pltpu.make_async_copy(el_sh.at[sidx_v.at[c]], elb, seme).wait()
        pltpu.make_async_copy(er_sh.at[didx_v.at[c]], erb, seme).wait()

        @pl.when(c >= 1)
        def _wait_den():
            pltpu.make_async_copy(wbuf, den_sh.at[didx_v.at[0]], semd).wait()

        @plsc.parallel_loop(0, CH * HEADS // 16, 1, unroll=4)
        def _wcalc(p):
            fl = p * 16 + i16
            rows = fl // HEADS
            cols = fl % HEADS
            z = (plsc.load_gather(elb, [rows, cols])
                 + plsc.load_gather(erb, [rows, cols]))
            wv = jnp.exp(jnp.where(z > 0, z, z * NEG_SLOPE))
            plsc.store_scatter(wbuf, [rows, cols], wv)

        pltpu.async_copy(wbuf, den_sh.at[didx_v.at[c]], semd, add=True)

        @pl.when(c + 1 < NCH)
        def _prefetch():
            pltpu.async_copy(el_sh.at[sidx_v.at[c + 1]], elbufs[1 - par], seme)
            pltpu.async_copy(er_sh.at[didx_v.at[c + 1]], erbufs[1 - par], seme)

        pltpu.make_async_copy(feat_hbm.at[sidx_v.at[c]], featbuf, semf).wait()

        @pl.when(c >= 1)
        def _wait_rst():
            pltpu.make_async_copy(msgbuf, rst_sh.at[didx_v.at[0]], semr).wait()

        @plsc.parallel_loop(0, CH // 2, 1, unroll=2)
        def _scale(p):
            fl = p * 16 + i16
            wrow = plsc.load_gather(wbuf, [fl // HEADS, fl % HEADS])
            for h in range(HEADS):
                sl = pl.ds(h * DH, DH)
                msgbuf[2 * p, sl] = featbuf[2 * p, sl] * wrow[h]
                msgbuf[2 * p + 1, sl] = featbuf[2 * p + 1, sl] * wrow[HEADS + h]

        @pl.when(c + 1 < NCH)
        def _prefetch_feat():
            pltpu.async_copy(feat_hbm.at[sidx_v.at[c + 1]], featbuf, semf)

        pltpu.async_copy(msgbuf, rst_sh.at[didx_v.at[c]], semr, add=True)

    def _chunk2(cc, _):
        _chunk(2 * cc, 0)
        _chunk(2 * cc + 1, 1)
        return 0

    lax.fori_loop(0, NCH // 2, _chunk2, 0)
    pltpu.make_async_copy(wbuf, den_sh.at[didx_v.at[0]], semd).wait()
    pltpu.make_async_copy(msgbuf, rst_sh.at[didx_v.at[0]], semr).wait()
    plsc.subcore_barrier()

    pltpu.sync_copy(rst_sh.at[pl.ds(base, RPT)],
                    rst_out.at[cid, pl.ds(base, RPT)])

    pltpu.sync_copy(den_sh.at[pl.ds(base, RPT)], den_v)
    for g in range(3):
        rows = CH if g < 2 else RPT - 2 * CH

        @plsc.parallel_loop(0, rows, 1, unroll=2)
        def _exp(r):
            for h in range(HEADS):
                val = plsc.load_gather(
                    den_v, [jnp.full((16,), g * CH + r, jnp.int32),
                            jnp.full((16,), h, jnp.int32)])
                featbuf[r, pl.ds(h * DH, DH)] = val
        pltpu.sync_copy(featbuf.at[pl.ds(0, rows)],
                        denx_out.at[cid, pl.ds(base + g * CH, rows)])


def _sc_pass(elr, feat, sidx, didx):
    mesh = plsc.VectorSubcoreMesh(core_axis_name="c", subcore_axis_name="s")
    fn = functools.partial(
        pl.kernel,
        mesh=mesh,
        compiler_params=pltpu.CompilerParams(needs_layout_passes=False,
                                             use_tc_tiling_on_sc=False),
        out_type=[
            jax.ShapeDtypeStruct((2, PD, HIDDEN), jnp.float32),
            jax.ShapeDtypeStruct((2, PD, HIDDEN), jnp.float32),
        ],
        scratch_types=[
            pltpu.VMEM((NCH, CH), jnp.int32),
            pltpu.VMEM((NCH, CH), jnp.int32),
            pltpu.VMEM((CH, HIDDEN), jnp.float32),
            pltpu.VMEM((CH, HIDDEN), jnp.float32),
            pltpu.VMEM((CH, HEADS), jnp.float32),
            pltpu.VMEM((CH, HEADS), jnp.float32),
            pltpu.VMEM((CH, HEADS), jnp.float32),
            pltpu.VMEM((CH, HEADS), jnp.float32),
            pltpu.VMEM((CH, HEADS), jnp.float32),
            pltpu.VMEM((RPT, HEADS), jnp.float32),
            pltpu.VMEM_SHARED((N_SRC, HEADS), jnp.float32),
            pltpu.VMEM_SHARED((PD, HEADS), jnp.float32),
            pltpu.VMEM_SHARED((PD, HIDDEN), jnp.float32),
            pltpu.VMEM_SHARED((PD, HEADS), jnp.float32),
            pltpu.SemaphoreType.DMA,
            pltpu.SemaphoreType.DMA,
            pltpu.SemaphoreType.DMA,
            pltpu.SemaphoreType.DMA,
        ],
    )(_sc_body)
    return fn(elr, feat, sidx, didx)


def _combine_body(r0_ref, r1_ref, d0_ref, d1_ref, elr_ref, feat_ref,
                  bias_ref, out_ref):
    el = elr_ref[:, 0:HEADS]
    er = elr_ref[:, HEADS:2 * HEADS]
    zs = el + er
    wself = jnp.exp(jnp.where(zs > 0, zs, zs * NEG_SLOPE))
    b = bias_ref[...]
    for h in range(HEADS):
        sl = slice(h * DH, (h + 1) * DH)
        wcol = wself[:, h:h + 1]
        num = r0_ref[0][:, sl] + r1_ref[0][:, sl] + wcol * feat_ref[:, sl]
        den = d0_ref[0][:, sl] + d1_ref[0][:, sl] + wcol + 1e-9
        out_ref[:, sl] = num / den + b[:, sl]


def _combine(rstp, denxp, elr, feat, bias2d):
    blk = 1000
    return pl.pallas_call(
        _combine_body,
        grid=(N_SRC // blk,),
        in_specs=[
            pl.BlockSpec((1, blk, HIDDEN), lambda i: (0, i, 0)),
            pl.BlockSpec((1, blk, HIDDEN), lambda i: (1, i, 0)),
            pl.BlockSpec((1, blk, HIDDEN), lambda i: (0, i, 0)),
            pl.BlockSpec((1, blk, HIDDEN), lambda i: (1, i, 0)),
            pl.BlockSpec((blk, HIDDEN), lambda i: (i + N_SRC // blk, 0)),
            pl.BlockSpec((blk, HIDDEN), lambda i: (i + N_SRC // blk, 0)),
            pl.BlockSpec((1, HIDDEN), lambda i: (0, 0)),
        ],
        out_specs=pl.BlockSpec((blk, HIDDEN), lambda i: (i, 0)),
        out_shape=jax.ShapeDtypeStruct((N_SRC, HIDDEN), jnp.float32),
    )(rstp, rstp, denxp, denxp, elr, feat, bias2d)


def kernel(x, edge_index, W, attn_l, attn_r, bias):
    alf = attn_l.reshape(HIDDEN)
    arf = attn_r.reshape(HIDDEN)
    sel = (jnp.arange(HIDDEN)[:, None] // DH
           == jnp.arange(HEADS)[None, :]).astype(jnp.float32)
    ALR = jnp.concatenate([alf[:, None] * sel, arf[:, None] * sel], axis=1)
    ALR = jnp.pad(ALR, ((0, 0), (0, HIDDEN - 2 * HEADS)))

    feat, elr = _dense(x, W, ALR)

    ei3 = edge_index.astype(jnp.int32).reshape(2, N_EDGES // 128, 128)
    sidx, didx = _edges(ei3)

    rstp, denxp = _sc_pass(elr, feat, sidx, didx)

    bias2d = bias.reshape(1, HIDDEN)
    return _combine(rstp, denxp, elr, feat, bias2d)
